# baseline (speedup 1.0000x reference)
import functools

import jax
import jax.numpy as jnp
import numpy as np
from jax import lax
from jax.experimental import pallas as pl
from jax.experimental.pallas import tpu as pltpu
from jax.experimental.pallas import tpu_sc as plsc

_B, _N, _E, _F, _D, _K, _V = 8, 1024, 3072, 4, 256, 16, 8192
_T = _N + _E
_BN = _B * _N
_L = 16
_C = 32
_NNODE = _N // 4
_NEDGE = _E // 4
_NCH_N = _NNODE // _C
_NCHUNK = (_NNODE + _NEDGE) // _C
_NPAIR = _NCHUNK // 2
_RB = 1024

_PERM = np.arange(_D).reshape(_D // 32, 2, _L).transpose(0, 2, 1).reshape(-1)
_PMAT = (np.arange(_D)[:, None] == _PERM[None, :]).astype(np.float32)


def _interleave_cols(a):
    r = a.shape[0]
    return a.reshape(r, _D // 32, 2, _L).swapaxes(2, 3).reshape(r, _D)


def _tbl_body(eig_ref, w_ref, o_ref, emb_ref, pm_ref, out_ref):
    x = eig_ref[...]
    w0 = w_ref[0:_K, :]
    w1 = w_ref[_K:2 * _K, :]
    p0 = jnp.dot(x, w0, preferred_element_type=jnp.float32)
    p1 = jnp.dot(x, w1, preferred_element_type=jnp.float32)
    o0 = o_ref[0:1, :]
    o1 = o_ref[1:2, :]
    out_ref[0] = jnp.dot(emb_ref[...], pm_ref[...],
                         preferred_element_type=jnp.float32
                         ).astype(jnp.bfloat16)
    out_ref[1] = p0.astype(jnp.bfloat16)
    out_ref[2] = (p1 + o0).astype(jnp.bfloat16)
    out_ref[3] = (p0 + p1 + o1).astype(jnp.bfloat16)
    out_ref[4] = (p1 + o1).astype(jnp.bfloat16)


def _build_table(lap_eigvec, lap_W, order_table, emb_table):
    out = pl.pallas_call(
        _tbl_body,
        grid=(_BN // _RB,),
        in_specs=[
            pl.BlockSpec((_RB, _K), lambda i: (i, 0)),
            pl.BlockSpec((2 * _K, _D), lambda i: (0, 0)),
            pl.BlockSpec((2, _D), lambda i: (0, 0)),
            pl.BlockSpec((_RB, _D), lambda i: (i, 0)),
            pl.BlockSpec((_D, _D), lambda i: (0, 0)),
        ],
        out_specs=pl.BlockSpec((5, _RB, _D), lambda i: (0, i, 0)),
        out_shape=jax.ShapeDtypeStruct((5, _BN, _D), jnp.bfloat16),
    )(lap_eigvec, lap_W, order_table, emb_table, jnp.asarray(_PMAT))
    return out.reshape(5 * _BN, _D)


def _sc_body(nd_ref, ed_ref, src_ref, dst_ref, tb_ref, out_ref,
             ia_n, ia_e, ibc, itmp,
             buf_a, buf_bc, outv,
             sem_i, sem_a, sem_b, sem_o):
    cid = lax.axis_index("c")
    sid = lax.axis_index("s")
    wid = sid * 2 + cid
    b = wid // 4
    q = wid % 4

    ntok0 = b * _N + q * _NNODE
    etok0 = b * _E + q * _NEDGE
    onode0 = b * _T + q * _NNODE
    oedge0 = b * _T + _N + q * _NEDGE

    cp1 = pltpu.async_copy(nd_ref.at[pl.ds(4 * ntok0, 4 * _NNODE)],
                           ia_n, sem_i)
    cp2 = pltpu.async_copy(ed_ref.at[pl.ds(4 * etok0, 4 * _NEDGE)],
                           ia_e, sem_i)
    cp3 = pltpu.async_copy(src_ref.at[pl.ds(etok0, _NEDGE)],
                           itmp.at[pl.ds(0, _NEDGE)], sem_i)
    cp4 = pltpu.async_copy(dst_ref.at[pl.ds(etok0, _NEDGE)],
                           itmp.at[pl.ds(_NEDGE, _NEDGE)], sem_i)
    cp1.wait()
    cp2.wait()
    cp3.wait()
    cp4.wait()

    def arith(k, carry):
        sl = pl.ds(k * _L, _L)
        sv = itmp[sl]
        dv = itmp[pl.ds(_NEDGE + k * _L, _L)]
        i = k // 2
        h = k % 2
        ibc[pl.ds(2 * _C * i + h * _L, _L)] = sv + (_BN + b * _N)
        ibc[pl.ds(2 * _C * i + _C + h * _L, _L)] = (
            dv + (2 * _BN + b * _N) + jnp.where(sv == dv, 2 * _BN, 0))
        return carry

    lax.fori_loop(0, _NEDGE // _L, arith, None)

    def gathers_node(i, slot):
        return [
            pltpu.make_async_copy(
                tb_ref.at[ia_n.at[pl.ds(i * 4 * _C, 4 * _C)]],
                buf_a[slot], sem_a[slot]),
            pltpu.make_async_copy(
                tb_ref.at[pl.ds(3 * _BN + ntok0 + i * _C, _C)],
                buf_bc[slot].at[pl.ds(0, _C)], sem_b[slot]),
        ]

    def gathers_edge(i, slot):
        e = i - _NCH_N
        return [
            pltpu.make_async_copy(
                tb_ref.at[ia_e.at[pl.ds(e * 4 * _C, 4 * _C)]],
                buf_a[slot], sem_a[slot]),
            pltpu.make_async_copy(
                tb_ref.at[ibc.at[pl.ds(e * 2 * _C, 2 * _C)]],
                buf_bc[slot], sem_b[slot]),
        ]

    def store(i, slot):
        obase = jnp.where(i < _NCH_N, onode0 + i * _C,
                          oedge0 + (i - _NCH_N) * _C)
        return pltpu.make_async_copy(
            outv[slot], out_ref.at[pl.ds(obase, _C)], sem_o[slot])

    def sum_rows(slot, with_c):
        def body(c, carry):
            for g in range(_D // 32):
                sl = pl.ds(_L * g, _L)
                lo = None
                hi = None
                rows = [buf_a[slot][4 * c, sl], buf_a[slot][4 * c + 1, sl],
                        buf_a[slot][4 * c + 2, sl], buf_a[slot][4 * c + 3, sl],
                        buf_bc[slot][c, sl]]
                if with_c:
                    rows.append(buf_bc[slot][_C + c, sl])
                for r in rows:
                    a = lax.bitcast_convert_type(r << 16, jnp.float32)
                    bb = lax.bitcast_convert_type(r & jnp.int32(-65536),
                                                  jnp.float32)
                    lo = a if lo is None else lo + a
                    hi = bb if hi is None else hi + bb
                outv[slot][c, pl.ds(32 * g, _L)] = lo
                outv[slot][c, pl.ds(32 * g + _L, _L)] = hi
            return carry
        lax.fori_loop(0, _C, body, None)

    def fire(i, slot):
        @pl.when(i < _NCH_N)
        def _():
            for cp in gathers_node(i, slot):
                cp.start()

        @pl.when(i >= _NCH_N)
        def _():
            for cp in gathers_edge(i, slot):
                cp.start()

    def consume(i, slot, s):
        @pl.when(i < _NCH_N)
        def _():
            for cp in gathers_node(i, slot):
                cp.wait()

        @pl.when(i >= _NCH_N)
        def _():
            for cp in gathers_edge(i, slot):
                cp.wait()

        @pl.when(s > 0)
        def _():
            store(i, slot).wait()

        @pl.when(i < _NCH_N)
        def _():
            sum_rows(slot, with_c=False)

        @pl.when(i >= _NCH_N)
        def _():
            sum_rows(slot, with_c=True)
        store(i, slot).start()

    fire(0, 0)

    def pair(s, carry):
        fire(2 * s + 1, 1)
        consume(2 * s, 0, s)

        @pl.when(s < _NPAIR - 1)
        def _():
            fire(2 * s + 2, 0)
        consume(2 * s + 1, 1, s)
        return carry

    lax.fori_loop(0, _NPAIR, pair, None)
    store(_NCHUNK - 2, 0).wait()
    store(_NCHUNK - 1, 1).wait()


def _gather_sum(nd_flat, ed_flat, src, dst, tb_words):
    mesh = plsc.VectorSubcoreMesh(core_axis_name="c", subcore_axis_name="s")
    fn = pl.kernel(
        _sc_body,
        out_type=jax.ShapeDtypeStruct((_B * _T, _D), jnp.float32),
        mesh=mesh,
        scratch_types=[
            pltpu.VMEM((4 * _NNODE,), jnp.int32),
            pltpu.VMEM((4 * _NEDGE,), jnp.int32),
            pltpu.VMEM((2 * _NEDGE,), jnp.int32),
            pltpu.VMEM((2 * _NEDGE,), jnp.int32),
            [pltpu.VMEM((4 * _C, _D // 2), jnp.int32)] * 2,
            [pltpu.VMEM((2 * _C, _D // 2), jnp.int32)] * 2,
            [pltpu.VMEM((_C, _D), jnp.float32)] * 2,
            pltpu.SemaphoreType.DMA,
            [pltpu.SemaphoreType.DMA] * 2,
            [pltpu.SemaphoreType.DMA] * 2,
            [pltpu.SemaphoreType.DMA] * 2,
        ],
    )
    return fn(nd_flat, ed_flat, src, dst, tb_words)


def _as_words(x_bf16):
    r = x_bf16.shape[0]
    return lax.bitcast_convert_type(
        x_bf16.reshape(r, _D // 2, 2), jnp.int32)


def kernel(node_data, edge_data, edge_index, lap_eigvec, emb_table, lap_W,
           order_table):
    tbig = _build_table(lap_eigvec, _interleave_cols(lap_W),
                        _interleave_cols(order_table), emb_table)
    feat = _gather_sum(node_data.reshape(-1), edge_data.reshape(-1),
                       edge_index[0], edge_index[1], _as_words(tbig))
    padded_feature = feat.reshape(_B, _T, _D)
    node_part = jnp.broadcast_to(
        jnp.arange(_N, dtype=edge_index.dtype)[None, :, None], (_B, _N, 2))
    edge_part = jnp.transpose(edge_index).reshape(_B, _E, 2)
    padded_index = jnp.concatenate([node_part, edge_part], axis=1)
    padding_mask = jnp.zeros((_B, _T), dtype=jnp.bool_)
    return padded_feature, padding_mask, padded_index


# speedup vs baseline: 1.6094x; 1.6094x over previous
import functools

import jax
import jax.numpy as jnp
from jax import lax
from jax.experimental import pallas as pl
from jax.experimental.pallas import tpu as pltpu
from jax.experimental.pallas import tpu_sc as plsc

_B, _N, _E, _F, _D, _K, _V = 8, 1024, 3072, 4, 256, 16, 8192
_T = _N + _E
_BN = _B * _N
_L = 16
_C = 32
_NNODE = _N // 4
_NEDGE = _E // 4
_NCH_N = _NNODE // _C
_NCHUNK = (_NNODE + _NEDGE) // _C
_NPAIR = _NCHUNK // 2
_RB = 1024


def _tbl_body(eig_ref, w_ref, o_ref, out_ref):
    x = eig_ref[...]
    w0 = w_ref[0:_K, :]
    w1 = w_ref[_K:2 * _K, :]
    p0 = jnp.dot(x, w0, preferred_element_type=jnp.float32)
    p1 = jnp.dot(x, w1, preferred_element_type=jnp.float32)
    o0 = o_ref[0:1, :]
    o1 = o_ref[1:2, :]
    out_ref[0] = p0
    out_ref[1] = p1 + o0
    out_ref[2] = p0 + p1 + o1
    out_ref[3] = p1 + o1


def _build_table(lap_eigvec, lap_W, order_table):
    out = pl.pallas_call(
        _tbl_body,
        grid=(_BN // _RB,),
        in_specs=[
            pl.BlockSpec((_RB, _K), lambda i: (i, 0)),
            pl.BlockSpec((2 * _K, _D), lambda i: (0, 0)),
            pl.BlockSpec((2, _D), lambda i: (0, 0)),
        ],
        out_specs=pl.BlockSpec((4, _RB, _D), lambda i: (0, i, 0)),
        out_shape=jax.ShapeDtypeStruct((4, _BN, _D), jnp.float32),
    )(lap_eigvec, lap_W, order_table)
    return out.reshape(4 * _BN, _D)


def _sc_body(nd_ref, ed_ref, src_ref, dst_ref, emb_ref, tb_ref, out_ref,
             ia_n, ia_e, ibc, itmp,
             buf_a, buf_bc, outv,
             sem_i, sem_a, sem_a2, sem_b, sem_o):
    cid = lax.axis_index("c")
    sid = lax.axis_index("s")
    wid = sid * 2 + cid
    b = wid // 4
    q = wid % 4

    ntok0 = b * _N + q * _NNODE
    etok0 = b * _E + q * _NEDGE
    onode0 = b * _T + q * _NNODE
    oedge0 = b * _T + _N + q * _NEDGE

    cp1 = pltpu.async_copy(nd_ref.at[pl.ds(4 * ntok0, 4 * _NNODE)],
                           ia_n, sem_i)
    cp2 = pltpu.async_copy(ed_ref.at[pl.ds(4 * etok0, 4 * _NEDGE)],
                           ia_e, sem_i)
    cp3 = pltpu.async_copy(src_ref.at[pl.ds(etok0, _NEDGE)],
                           itmp.at[pl.ds(0, _NEDGE)], sem_i)
    cp4 = pltpu.async_copy(dst_ref.at[pl.ds(etok0, _NEDGE)],
                           itmp.at[pl.ds(_NEDGE, _NEDGE)], sem_i)
    cp1.wait()
    cp2.wait()
    cp3.wait()
    cp4.wait()

    def arith(k, carry):
        sl = pl.ds(k * _L, _L)
        sv = itmp[sl]
        dv = itmp[pl.ds(_NEDGE + k * _L, _L)]
        i = k // 2
        h = k % 2
        ibc[pl.ds(2 * _C * i + h * _L, _L)] = sv + b * _N
        ibc[pl.ds(2 * _C * i + _C + h * _L, _L)] = (
            dv + (_BN + b * _N) + jnp.where(sv == dv, 2 * _BN, 0))
        return carry

    lax.fori_loop(0, _NEDGE // _L, arith, None)

    _H = 2 * _C
    def gathers_node(i, slot):
        return [
            pltpu.make_async_copy(
                emb_ref.at[ia_n.at[pl.ds(i * 4 * _C, _H)]],
                buf_a[slot].at[pl.ds(0, _H)], sem_a[slot]),
            pltpu.make_async_copy(
                emb_ref.at[ia_n.at[pl.ds(i * 4 * _C + _H, _H)]],
                buf_a[slot].at[pl.ds(_H, _H)], sem_a2[slot]),
            pltpu.make_async_copy(
                tb_ref.at[pl.ds(2 * _BN + ntok0 + i * _C, _C)],
                buf_bc[slot].at[pl.ds(0, _C)], sem_b[slot]),
        ]

    def gathers_edge(i, slot):
        e = i - _NCH_N
        return [
            pltpu.make_async_copy(
                emb_ref.at[ia_e.at[pl.ds(e * 4 * _C, _H)]],
                buf_a[slot].at[pl.ds(0, _H)], sem_a[slot]),
            pltpu.make_async_copy(
                emb_ref.at[ia_e.at[pl.ds(e * 4 * _C + _H, _H)]],
                buf_a[slot].at[pl.ds(_H, _H)], sem_a2[slot]),
            pltpu.make_async_copy(
                tb_ref.at[ibc.at[pl.ds(e * 2 * _C, 2 * _C)]],
                buf_bc[slot], sem_b[slot]),
        ]

    def store(i, slot):
        obase = jnp.where(i < _NCH_N, onode0 + i * _C,
                          oedge0 + (i - _NCH_N) * _C)
        return pltpu.make_async_copy(
            outv[slot], out_ref.at[pl.ds(obase, _C)], sem_o[slot])

    def sum_rows(slot, with_c):
        def body(c, carry):
            for h in range(_D // _L):
                sl = pl.ds(h * _L, _L)
                acc = (buf_a[slot][4 * c, sl] + buf_a[slot][4 * c + 1, sl]
                       + buf_a[slot][4 * c + 2, sl] + buf_a[slot][4 * c + 3, sl])
                acc = acc + buf_bc[slot][c, sl]
                if with_c:
                    acc = acc + buf_bc[slot][_C + c, sl]
                outv[slot][c, sl] = acc
            return carry
        lax.fori_loop(0, _C, body, None)

    def fire(i, slot):
        @pl.when(i < _NCH_N)
        def _():
            for cp in gathers_node(i, slot):
                cp.start()

        @pl.when(i >= _NCH_N)
        def _():
            for cp in gathers_edge(i, slot):
                cp.start()

    def consume(i, slot, s):
        @pl.when(i < _NCH_N)
        def _():
            for cp in gathers_node(i, slot):
                cp.wait()

        @pl.when(i >= _NCH_N)
        def _():
            for cp in gathers_edge(i, slot):
                cp.wait()

        @pl.when(s > 0)
        def _():
            store(i, slot).wait()

        @pl.when(i < _NCH_N)
        def _():
            sum_rows(slot, with_c=False)

        @pl.when(i >= _NCH_N)
        def _():
            sum_rows(slot, with_c=True)
        store(i, slot).start()

    fire(0, 0)

    def pair(s, carry):
        fire(2 * s + 1, 1)
        consume(2 * s, 0, s)

        @pl.when(s < _NPAIR - 1)
        def _():
            fire(2 * s + 2, 0)
        consume(2 * s + 1, 1, s)
        return carry

    lax.fori_loop(0, _NPAIR, pair, None)
    store(_NCHUNK - 2, 0).wait()
    store(_NCHUNK - 1, 1).wait()


def _gather_sum(nd_flat, ed_flat, src, dst, emb_table, tbig):
    mesh = plsc.VectorSubcoreMesh(core_axis_name="c", subcore_axis_name="s")
    fn = pl.kernel(
        _sc_body,
        out_type=jax.ShapeDtypeStruct((_B * _T, _D), jnp.float32),
        mesh=mesh,
        scratch_types=[
            pltpu.VMEM((4 * _NNODE,), jnp.int32),
            pltpu.VMEM((4 * _NEDGE,), jnp.int32),
            pltpu.VMEM((2 * _NEDGE,), jnp.int32),
            pltpu.VMEM((2 * _NEDGE,), jnp.int32),
            [pltpu.VMEM((4 * _C, _D), jnp.float32)] * 2,
            [pltpu.VMEM((2 * _C, _D), jnp.float32)] * 2,
            [pltpu.VMEM((_C, _D), jnp.float32)] * 2,
            pltpu.SemaphoreType.DMA,
            [pltpu.SemaphoreType.DMA] * 2,
            [pltpu.SemaphoreType.DMA] * 2,
            [pltpu.SemaphoreType.DMA] * 2,
            [pltpu.SemaphoreType.DMA] * 2,
        ],
    )
    return fn(nd_flat, ed_flat, src, dst, emb_table, tbig)


def kernel(node_data, edge_data, edge_index, lap_eigvec, emb_table, lap_W,
           order_table):
    tbig = _build_table(lap_eigvec, lap_W, order_table)
    feat = _gather_sum(node_data.reshape(-1), edge_data.reshape(-1),
                       edge_index[0], edge_index[1], emb_table, tbig)
    padded_feature = feat.reshape(_B, _T, _D)
    node_part = jnp.broadcast_to(
        jnp.arange(_N, dtype=edge_index.dtype)[None, :, None], (_B, _N, 2))
    edge_part = jnp.transpose(edge_index).reshape(_B, _E, 2)
    padded_index = jnp.concatenate([node_part, edge_part], axis=1)
    padding_mask = jnp.zeros((_B, _T), dtype=jnp.bool_)
    return padded_feature, padding_mask, padded_index
